# Initial kernel scaffold; baseline (speedup 1.0000x reference)
import jax
import jax.numpy as jnp
from jax.experimental import pallas as pl


def kernel(seqs, query_tok, embed, sW1, sb1, sW2, sb2, rW1, rb1, rW2, rb2):
    raise NotImplementedError("write your pallas kernel here")


# speedup vs baseline: 221.6581x; 221.6581x over previous
import functools

import jax
import jax.numpy as jnp
from jax import lax
from jax.experimental import pallas as pl
from jax.experimental.pallas import tpu as pltpu
from jax.experimental.pallas import tpu_sc as plsc

HID = 64
VOCAB = 64
SLOTS = 8
SEQ = 32
LANES = 16
NW = 32
OHW = 2 * VOCAB


def _prep_body(embed_ref, sW1_ref, sb1_ref, sW2_ref, sb2_ref, rW1_ref,
               ts_ref, wcat_ref):
    emb = embed_ref[0:VOCAB, :]
    h = jnp.maximum(jnp.dot(emb, sW1_ref[...]) + sb1_ref[...], 0.0)
    ts_ref[...] = jnp.dot(h, sW2_ref[...]) + sb2_ref[...]
    wcat_ref[0:VOCAB, :] = jnp.dot(emb, rW1_ref[0:HID, :])
    wcat_ref[VOCAB:OHW, :] = jnp.dot(emb, rW1_ref[HID:2 * HID, :]) * (1.0 / SLOTS)


def _prep(embed, sW1, sb1, sW2, sb2, rW1):
    return pl.pallas_call(
        _prep_body,
        out_shape=(
            jax.ShapeDtypeStruct((VOCAB, 1), jnp.float32),
            jax.ShapeDtypeStruct((OHW, HID), jnp.float32),
        ),
    )(embed, sW1, sb1, sW2, sb2, rW1)


def _sim_body(seqs_hbm, q_hbm, ts_hbm, oh_hbm, ts_v, seq_v, q_v, oh_v):
    wid = lax.axis_index("s") * 2 + lax.axis_index("c")
    per_w = q_hbm.shape[0] // NW
    groups = per_w // LANES

    pltpu.sync_copy(ts_hbm, ts_v)

    zeros_f = jnp.zeros((LANES,), jnp.float32)
    ones_f = jnp.full((LANES,), 1.0, jnp.float32)
    lane = lax.iota(jnp.int32, LANES)
    lane_seq = lane * SEQ
    lane_oh = lane * OHW

    for c in range(LANES * OHW // LANES):
        oh_v[pl.ds(c * LANES, LANES)] = zeros_f

    def group(g, carry):
        base = wid * per_w + g * LANES
        pltpu.sync_copy(seqs_hbm.at[pl.ds(base * SEQ, LANES * SEQ)], seq_v)
        pltpu.sync_copy(q_hbm.at[pl.ds(base, LANES)], q_v)
        qtok = q_v[...]

        tok = []
        sc = []
        for t in range(SLOTS):
            tt = plsc.load_gather(seq_v, [lane_seq + t])
            tok.append(tt)
            sc.append(plsc.load_gather(ts_v, [tt]))

        for t in range(SLOTS, SEQ - 1):
            ntok = plsc.load_gather(seq_v, [lane_seq + t])
            nsc = plsc.load_gather(ts_v, [ntok])
            m = sc[0]
            for i in range(1, SLOTS):
                m = jnp.minimum(m, sc[i])
            ev = jnp.full((LANES,), SLOTS - 1, jnp.int32)
            for i in range(SLOTS - 2, -1, -1):
                ev = jnp.where(sc[i] == m, jnp.full((LANES,), i, jnp.int32), ev)
            ntoks = []
            nscs = []
            for i in range(SLOTS - 1):
                keep = ev > i
                ntoks.append(jnp.where(keep, tok[i], tok[i + 1]))
                nscs.append(jnp.where(keep, sc[i], sc[i + 1]))
            tok = ntoks + [ntok]
            sc = nscs + [nsc]

        plsc.store_scatter(oh_v, [lane_oh + qtok], ones_f)
        for i in range(SLOTS):
            plsc.addupdate_scatter(oh_v, [lane_oh + (tok[i] + VOCAB)], ones_f)
        pltpu.sync_copy(oh_v, oh_hbm.at[pl.ds(base * OHW, LANES * OHW)])
        plsc.store_scatter(oh_v, [lane_oh + qtok], zeros_f)
        for i in range(SLOTS):
            plsc.store_scatter(oh_v, [lane_oh + (tok[i] + VOCAB)], zeros_f)
        return carry

    lax.fori_loop(0, groups, group, 0)


def _sim(seqs, query_tok, ts):
    B = query_tok.shape[0]
    mesh = plsc.VectorSubcoreMesh(core_axis_name="c", subcore_axis_name="s")
    f = functools.partial(
        pl.kernel,
        out_type=jax.ShapeDtypeStruct((B * OHW,), jnp.float32),
        mesh=mesh,
        scratch_types=[
            pltpu.VMEM((VOCAB,), jnp.float32),
            pltpu.VMEM((LANES * SEQ,), jnp.int32),
            pltpu.VMEM((LANES,), jnp.int32),
            pltpu.VMEM((LANES * OHW,), jnp.float32),
        ],
        compiler_params=pltpu.CompilerParams(needs_layout_passes=False),
    )(_sim_body)
    return f(seqs.reshape(-1), query_tok, ts).reshape(B, OHW)


def _head_body(oh_ref, wcat_ref, rb1_ref, rW2_ref, rb2_ref, out_ref):
    h = jnp.maximum(jnp.dot(oh_ref[...], wcat_ref[...]) + rb1_ref[...], 0.0)
    out_ref[...] = jnp.dot(h, rW2_ref[...]) + rb2_ref[...]


def _head(ohcat, wcat, rb1, rW2, rb2):
    B = ohcat.shape[0]
    blk = 2048
    return pl.pallas_call(
        _head_body,
        grid=(B // blk,),
        in_specs=[
            pl.BlockSpec((blk, OHW), lambda i: (i, 0)),
            pl.BlockSpec((OHW, HID), lambda i: (0, 0)),
            pl.BlockSpec((1, HID), lambda i: (0, 0)),
            pl.BlockSpec((HID, VOCAB), lambda i: (0, 0)),
            pl.BlockSpec((1, VOCAB), lambda i: (0, 0)),
        ],
        out_specs=pl.BlockSpec((blk, VOCAB), lambda i: (i, 0)),
        out_shape=jax.ShapeDtypeStruct((B, VOCAB), jnp.float32),
        compiler_params=pltpu.CompilerParams(
            dimension_semantics=("parallel",)),
    )(ohcat, wcat, rb1, rW2, rb2)


def kernel(seqs, query_tok, embed, sW1, sb1, sW2, sb2, rW1, rb1, rW2, rb2):
    seqs = seqs.astype(jnp.int32)
    query_tok = query_tok.astype(jnp.int32)
    ts2d, wcat = _prep(embed, sW1, sb1.reshape(1, -1), sW2, sb2.reshape(1, -1), rW1)
    ohcat = _sim(seqs, query_tok, ts2d.reshape(VOCAB))
    return _head(ohcat, wcat, rb1.reshape(1, -1), rW2, rb2.reshape(1, -1))


# speedup vs baseline: 280.9764x; 1.2676x over previous
import functools

import jax
import jax.numpy as jnp
from jax import lax
from jax.experimental import pallas as pl
from jax.experimental.pallas import tpu as pltpu
from jax.experimental.pallas import tpu_sc as plsc

HID = 64
VOCAB = 64
SLOTS = 8
SEQ = 32
LANES = 16
NW = 32
OHW = 2 * VOCAB


def _prep_body(embed_ref, sW1_ref, sb1_ref, sW2_ref, sb2_ref, rW1_ref,
               ts_ref, wcat_ref):
    emb = embed_ref[0:VOCAB, :]
    h = jnp.maximum(jnp.dot(emb, sW1_ref[...]) + sb1_ref[...], 0.0)
    ts_ref[...] = jnp.dot(h, sW2_ref[...]) + sb2_ref[...]
    wcat_ref[0:VOCAB, :] = jnp.dot(emb, rW1_ref[0:HID, :])
    wcat_ref[VOCAB:OHW, :] = jnp.dot(emb, rW1_ref[HID:2 * HID, :]) * (1.0 / SLOTS)


def _prep(embed, sW1, sb1, sW2, sb2, rW1):
    return pl.pallas_call(
        _prep_body,
        out_shape=(
            jax.ShapeDtypeStruct((VOCAB, 1), jnp.float32),
            jax.ShapeDtypeStruct((OHW, HID), jnp.float32),
        ),
    )(embed, sW1, sb1, sW2, sb2, rW1)


def _sim_body(seqs_hbm, ts_hbm, cnt_hbm, ts_v, seq_v, cnt_v):
    wid = lax.axis_index("s") * 2 + lax.axis_index("c")
    per_w = seqs_hbm.shape[0] // SEQ // NW
    groups = per_w // LANES
    base = wid * per_w

    pltpu.sync_copy(ts_hbm, ts_v)
    pltpu.sync_copy(seqs_hbm.at[pl.ds(base * SEQ, per_w * SEQ)], seq_v)

    zeros_f = jnp.zeros((LANES,), jnp.float32)
    ones_f = jnp.full((LANES,), 1.0, jnp.float32)
    lane = lax.iota(jnp.int32, LANES)
    lane_seq = lane * SEQ
    lane_cnt = lane * VOCAB
    iconsts = [jnp.full((LANES,), i, jnp.int32) for i in range(SLOTS)]

    def group(g, carry):
        goff_seq = lane_seq + g * (LANES * SEQ)
        goff_cnt = lane_cnt + g * (LANES * VOCAB)

        for k in range(LANES * VOCAB // LANES):
            cnt_v[pl.ds(g * (LANES * VOCAB) + k * LANES, LANES)] = zeros_f

        tok = []
        sc = []
        for t in range(SLOTS):
            tt = plsc.load_gather(seq_v, [goff_seq + t])
            tok.append(tt)
            sc.append(plsc.load_gather(ts_v, [tt]))

        for t in range(SLOTS, SEQ - 1):
            ntok = plsc.load_gather(seq_v, [goff_seq + t])
            nsc = plsc.load_gather(ts_v, [ntok])
            m01 = jnp.minimum(sc[0], sc[1])
            m23 = jnp.minimum(sc[2], sc[3])
            m45 = jnp.minimum(sc[4], sc[5])
            m67 = jnp.minimum(sc[6], sc[7])
            m = jnp.minimum(jnp.minimum(m01, m23), jnp.minimum(m45, m67))
            cand = [jnp.where(sc[i] == m, iconsts[i], iconsts[SLOTS - 1])
                    for i in range(SLOTS - 1)]
            c01 = jnp.minimum(cand[0], cand[1])
            c23 = jnp.minimum(cand[2], cand[3])
            c45 = jnp.minimum(cand[4], cand[5])
            ev = jnp.minimum(jnp.minimum(c01, c23),
                             jnp.minimum(c45, cand[6]))
            ntoks = []
            nscs = []
            for i in range(SLOTS - 1):
                keep = ev > i
                ntoks.append(jnp.where(keep, tok[i], tok[i + 1]))
                nscs.append(jnp.where(keep, sc[i], sc[i + 1]))
            tok = ntoks + [ntok]
            sc = nscs + [nsc]

        for i in range(SLOTS):
            plsc.addupdate_scatter(cnt_v, [goff_cnt + tok[i]], ones_f)
        return carry

    lax.fori_loop(0, groups, group, 0)
    pltpu.sync_copy(cnt_v, cnt_hbm.at[pl.ds(base * VOCAB, per_w * VOCAB)])


def _sim(seqs, ts):
    B = seqs.shape[0]
    mesh = plsc.VectorSubcoreMesh(core_axis_name="c", subcore_axis_name="s")
    per_w = B // NW
    f = functools.partial(
        pl.kernel,
        out_type=jax.ShapeDtypeStruct((B * VOCAB,), jnp.float32),
        mesh=mesh,
        scratch_types=[
            pltpu.VMEM((VOCAB,), jnp.float32),
            pltpu.VMEM((per_w * SEQ,), jnp.int32),
            pltpu.VMEM((per_w * VOCAB,), jnp.float32),
        ],
        compiler_params=pltpu.CompilerParams(needs_layout_passes=False),
    )(_sim_body)
    return f(seqs.reshape(-1), ts).reshape(B, VOCAB)


def _head_body(cnt_ref, q_ref, wcat_ref, rb1_ref, rW2_ref, rb2_ref, out_ref):
    blk = cnt_ref.shape[0]
    iota = lax.broadcasted_iota(jnp.int32, (blk, VOCAB), 1)
    qoh = (q_ref[...] == iota).astype(jnp.float32)
    h = (jnp.dot(qoh, wcat_ref[0:VOCAB, :])
         + jnp.dot(cnt_ref[...], wcat_ref[VOCAB:OHW, :])
         + rb1_ref[...])
    h = jnp.maximum(h, 0.0)
    out_ref[...] = jnp.dot(h, rW2_ref[...]) + rb2_ref[...]


def _head(counts, query_tok, wcat, rb1, rW2, rb2):
    B = counts.shape[0]
    blk = 2048
    return pl.pallas_call(
        _head_body,
        grid=(B // blk,),
        in_specs=[
            pl.BlockSpec((blk, VOCAB), lambda i: (i, 0)),
            pl.BlockSpec((blk, 1), lambda i: (i, 0)),
            pl.BlockSpec((OHW, HID), lambda i: (0, 0)),
            pl.BlockSpec((1, HID), lambda i: (0, 0)),
            pl.BlockSpec((HID, VOCAB), lambda i: (0, 0)),
            pl.BlockSpec((1, VOCAB), lambda i: (0, 0)),
        ],
        out_specs=pl.BlockSpec((blk, VOCAB), lambda i: (i, 0)),
        out_shape=jax.ShapeDtypeStruct((B, VOCAB), jnp.float32),
        compiler_params=pltpu.CompilerParams(
            dimension_semantics=("parallel",)),
    )(counts, query_tok.reshape(B, 1), wcat, rb1, rW2, rb2)


def kernel(seqs, query_tok, embed, sW1, sb1, sW2, sb2, rW1, rb1, rW2, rb2):
    seqs = seqs.astype(jnp.int32)
    query_tok = query_tok.astype(jnp.int32)
    ts2d, wcat = _prep(embed, sW1, sb1.reshape(1, -1), sW2, sb2.reshape(1, -1), rW1)
    counts = _sim(seqs, ts2d.reshape(VOCAB))
    return _head(counts, query_tok, wcat, rb1.reshape(1, -1), rW2, rb2.reshape(1, -1))


# speedup vs baseline: 301.0317x; 1.0714x over previous
import functools

import jax
import jax.numpy as jnp
from jax import lax
from jax.experimental import pallas as pl
from jax.experimental.pallas import tpu as pltpu
from jax.experimental.pallas import tpu_sc as plsc

HID = 64
VOCAB = 64
SLOTS = 8
SEQ = 32
LANES = 16
NW = 32
OHW = 2 * VOCAB


def _prep_body(embed_ref, sW1_ref, sb1_ref, sW2_ref, sb2_ref, rW1_ref,
               ts_ref, wcat_ref):
    emb = embed_ref[0:VOCAB, :]
    h = jnp.maximum(jnp.dot(emb, sW1_ref[...]) + sb1_ref[...], 0.0)
    ts_ref[...] = jnp.dot(h, sW2_ref[...]) + sb2_ref[...]
    wcat_ref[0:VOCAB, :] = jnp.dot(emb, rW1_ref[0:HID, :])
    wcat_ref[VOCAB:OHW, :] = jnp.dot(emb, rW1_ref[HID:2 * HID, :]) * (1.0 / SLOTS)


def _prep(embed, sW1, sb1, sW2, sb2, rW1):
    return pl.pallas_call(
        _prep_body,
        out_shape=(
            jax.ShapeDtypeStruct((VOCAB, 1), jnp.float32),
            jax.ShapeDtypeStruct((OHW, HID), jnp.float32),
        ),
    )(embed, sW1, sb1, sW2, sb2, rW1)


def _sim_body(seqs_hbm, ts_hbm, cnt_hbm, ts_v, seq_v, cnt_v):
    wid = lax.axis_index("s") * 2 + lax.axis_index("c")
    per_w = seqs_hbm.shape[0] // SEQ // NW
    groups = per_w // LANES
    base = wid * per_w

    pltpu.sync_copy(ts_hbm, ts_v)
    pltpu.sync_copy(seqs_hbm.at[pl.ds(base * SEQ, per_w * SEQ)], seq_v)

    zeros_f = jnp.zeros((LANES,), jnp.float32)
    ones_f = jnp.full((LANES,), 1.0, jnp.float32)
    lane = lax.iota(jnp.int32, LANES)
    lane_seq = lane * SEQ
    lane_cnt = lane * VOCAB

    def one_group(g):
        goff_seq = lane_seq + g * (LANES * SEQ)
        goff_cnt = lane_cnt + g * (LANES * VOCAB)

        for k in range(LANES * VOCAB // LANES):
            cnt_v[pl.ds(g * (LANES * VOCAB) + k * LANES, LANES)] = zeros_f

        tok = []
        sc = []
        for t in range(SLOTS):
            tt = plsc.load_gather(seq_v, [goff_seq + t])
            tok.append(tt)
            sc.append(plsc.load_gather(ts_v, [tt]))

        for t in range(SLOTS, SEQ - 1):
            ntok = plsc.load_gather(seq_v, [goff_seq + t])
            nsc = plsc.load_gather(ts_v, [ntok])
            pref = [sc[0]]
            for i in range(1, SLOTS - 1):
                pref.append(jnp.minimum(pref[-1], sc[i]))
            m = jnp.minimum(pref[-1], sc[SLOTS - 1])
            ntoks = []
            nscs = []
            for i in range(SLOTS - 1):
                keep = pref[i] > m
                ntoks.append(jnp.where(keep, tok[i], tok[i + 1]))
                nscs.append(jnp.where(keep, sc[i], sc[i + 1]))
            tok = ntoks + [ntok]
            sc = nscs + [nsc]

        for i in range(SLOTS):
            plsc.addupdate_scatter(cnt_v, [goff_cnt + tok[i]], ones_f)

    def group_pair(gg, carry):
        one_group(gg * 2)
        one_group(gg * 2 + 1)
        return carry

    lax.fori_loop(0, groups // 2, group_pair, 0)
    pltpu.sync_copy(cnt_v, cnt_hbm.at[pl.ds(base * VOCAB, per_w * VOCAB)])


def _sim(seqs, ts):
    B = seqs.shape[0]
    mesh = plsc.VectorSubcoreMesh(core_axis_name="c", subcore_axis_name="s")
    per_w = B // NW
    f = functools.partial(
        pl.kernel,
        out_type=jax.ShapeDtypeStruct((B * VOCAB,), jnp.float32),
        mesh=mesh,
        scratch_types=[
            pltpu.VMEM((VOCAB,), jnp.float32),
            pltpu.VMEM((per_w * SEQ,), jnp.int32),
            pltpu.VMEM((per_w * VOCAB,), jnp.float32),
        ],
        compiler_params=pltpu.CompilerParams(needs_layout_passes=False),
    )(_sim_body)
    return f(seqs.reshape(-1), ts).reshape(B, VOCAB)


def _head_body(cnt_ref, q_ref, wcat_ref, rb1_ref, rW2_ref, rb2_ref, out_ref):
    blk = cnt_ref.shape[0]
    iota = lax.broadcasted_iota(jnp.int32, (blk, VOCAB), 1)
    qoh = (q_ref[...] == iota).astype(jnp.float32)
    h = (jnp.dot(qoh, wcat_ref[0:VOCAB, :])
         + jnp.dot(cnt_ref[...], wcat_ref[VOCAB:OHW, :])
         + rb1_ref[...])
    h = jnp.maximum(h, 0.0)
    out_ref[...] = jnp.dot(h, rW2_ref[...]) + rb2_ref[...]


def _head(counts, query_tok, wcat, rb1, rW2, rb2):
    B = counts.shape[0]
    blk = 2048
    return pl.pallas_call(
        _head_body,
        grid=(B // blk,),
        in_specs=[
            pl.BlockSpec((blk, VOCAB), lambda i: (i, 0)),
            pl.BlockSpec((blk, 1), lambda i: (i, 0)),
            pl.BlockSpec((OHW, HID), lambda i: (0, 0)),
            pl.BlockSpec((1, HID), lambda i: (0, 0)),
            pl.BlockSpec((HID, VOCAB), lambda i: (0, 0)),
            pl.BlockSpec((1, VOCAB), lambda i: (0, 0)),
        ],
        out_specs=pl.BlockSpec((blk, VOCAB), lambda i: (i, 0)),
        out_shape=jax.ShapeDtypeStruct((B, VOCAB), jnp.float32),
        compiler_params=pltpu.CompilerParams(
            dimension_semantics=("parallel",)),
    )(counts, query_tok.reshape(B, 1), wcat, rb1, rW2, rb2)


def kernel(seqs, query_tok, embed, sW1, sb1, sW2, sb2, rW1, rb1, rW2, rb2):
    seqs = seqs.astype(jnp.int32)
    query_tok = query_tok.astype(jnp.int32)
    ts2d, wcat = _prep(embed, sW1, sb1.reshape(1, -1), sW2, sb2.reshape(1, -1), rW1)
    counts = _sim(seqs, ts2d.reshape(VOCAB))
    return _head(counts, query_tok, wcat, rb1.reshape(1, -1), rW2, rb2.reshape(1, -1))


# speedup vs baseline: 307.3206x; 1.0209x over previous
import functools

import jax
import jax.numpy as jnp
from jax import lax
from jax.experimental import pallas as pl
from jax.experimental.pallas import tpu as pltpu
from jax.experimental.pallas import tpu_sc as plsc

HID = 64
VOCAB = 64
SLOTS = 8
SEQ = 32
LANES = 16
NW = 32
OHW = 2 * VOCAB


def _prep_body(embed_ref, sW1_ref, sb1_ref, sW2_ref, sb2_ref, rW1_ref,
               ts_ref, wcat_ref):
    emb = embed_ref[0:VOCAB, :]
    h = jnp.maximum(jnp.dot(emb, sW1_ref[...]) + sb1_ref[...], 0.0)
    ts_ref[...] = jnp.dot(h, sW2_ref[...]) + sb2_ref[...]
    wcat_ref[0:VOCAB, :] = jnp.dot(emb, rW1_ref[0:HID, :])
    wcat_ref[VOCAB:OHW, :] = jnp.dot(emb, rW1_ref[HID:2 * HID, :]) * (1.0 / SLOTS)


def _prep(embed, sW1, sb1, sW2, sb2, rW1):
    return pl.pallas_call(
        _prep_body,
        out_shape=(
            jax.ShapeDtypeStruct((VOCAB, 1), jnp.float32),
            jax.ShapeDtypeStruct((OHW, HID), jnp.float32),
        ),
    )(embed, sW1, sb1, sW2, sb2, rW1)


def _sim_body(seqs_hbm, ts_hbm, cnt_hbm, ts_v, seq_v, cnt_v):
    wid = lax.axis_index("s") * 2 + lax.axis_index("c")
    per_w = seqs_hbm.shape[0] // NW
    groups = per_w // LANES
    base = wid * per_w

    pltpu.sync_copy(ts_hbm, ts_v)
    pltpu.sync_copy(seqs_hbm.at[pl.ds(base, per_w), :], seq_v)

    zeros_f = jnp.zeros((LANES,), jnp.float32)
    ones_f = jnp.full((LANES,), 1.0, jnp.float32)
    lane = lax.iota(jnp.int32, LANES)

    def one_group(g):
        samp = lane + g * LANES

        for r in range(LANES):
            for k in range(VOCAB // LANES):
                cnt_v[g * LANES + r, pl.ds(k * LANES, LANES)] = zeros_f

        tok = []
        sc = []
        for t in range(SLOTS):
            tt = plsc.load_gather(seq_v, [samp, jnp.full((LANES,), t, jnp.int32)])
            tok.append(tt)
            sc.append(plsc.load_gather(ts_v, [tt]))

        for t in range(SLOTS, SEQ - 1):
            ntok = plsc.load_gather(seq_v, [samp, jnp.full((LANES,), t, jnp.int32)])
            nsc = plsc.load_gather(ts_v, [ntok])
            pref = [sc[0]]
            for i in range(1, SLOTS - 1):
                pref.append(jnp.minimum(pref[-1], sc[i]))
            m = jnp.minimum(pref[-1], sc[SLOTS - 1])
            ntoks = []
            nscs = []
            for i in range(SLOTS - 1):
                keep = pref[i] > m
                ntoks.append(jnp.where(keep, tok[i], tok[i + 1]))
                nscs.append(jnp.where(keep, sc[i], sc[i + 1]))
            tok = ntoks + [ntok]
            sc = nscs + [nsc]

        for i in range(SLOTS):
            plsc.addupdate_scatter(cnt_v, [samp, tok[i]], ones_f)

    def group_pair(gg, carry):
        one_group(gg * 2)
        one_group(gg * 2 + 1)
        return carry

    lax.fori_loop(0, groups // 2, group_pair, 0)
    pltpu.sync_copy(cnt_v, cnt_hbm.at[pl.ds(base, per_w), :])


def _sim(seqs, ts):
    B = seqs.shape[0]
    mesh = plsc.VectorSubcoreMesh(core_axis_name="c", subcore_axis_name="s")
    per_w = B // NW
    f = functools.partial(
        pl.kernel,
        out_type=jax.ShapeDtypeStruct((B, VOCAB), jnp.float32),
        mesh=mesh,
        scratch_types=[
            pltpu.VMEM((VOCAB,), jnp.float32),
            pltpu.VMEM((per_w, SEQ), jnp.int32),
            pltpu.VMEM((per_w, VOCAB), jnp.float32),
        ],
        compiler_params=pltpu.CompilerParams(
            needs_layout_passes=False, use_tc_tiling_on_sc=False),
    )(_sim_body)
    return f(seqs, ts)


def _head_body(cnt_ref, q_ref, wcat_ref, rb1_ref, rW2_ref, rb2_ref, out_ref):
    blk = cnt_ref.shape[0]
    iota = lax.broadcasted_iota(jnp.int32, (blk, VOCAB), 1)
    qoh = (q_ref[...] == iota).astype(jnp.float32)
    h = (jnp.dot(qoh, wcat_ref[0:VOCAB, :])
         + jnp.dot(cnt_ref[...], wcat_ref[VOCAB:OHW, :])
         + rb1_ref[...])
    h = jnp.maximum(h, 0.0)
    out_ref[...] = jnp.dot(h, rW2_ref[...]) + rb2_ref[...]


def _head(counts, query_tok, wcat, rb1, rW2, rb2):
    B = counts.shape[0]
    blk = 2048
    return pl.pallas_call(
        _head_body,
        grid=(B // blk,),
        in_specs=[
            pl.BlockSpec((blk, VOCAB), lambda i: (i, 0)),
            pl.BlockSpec((blk, 1), lambda i: (i, 0)),
            pl.BlockSpec((OHW, HID), lambda i: (0, 0)),
            pl.BlockSpec((1, HID), lambda i: (0, 0)),
            pl.BlockSpec((HID, VOCAB), lambda i: (0, 0)),
            pl.BlockSpec((1, VOCAB), lambda i: (0, 0)),
        ],
        out_specs=pl.BlockSpec((blk, VOCAB), lambda i: (i, 0)),
        out_shape=jax.ShapeDtypeStruct((B, VOCAB), jnp.float32),
        compiler_params=pltpu.CompilerParams(
            dimension_semantics=("parallel",)),
    )(counts, query_tok.reshape(B, 1), wcat, rb1, rW2, rb2)


def kernel(seqs, query_tok, embed, sW1, sb1, sW2, sb2, rW1, rb1, rW2, rb2):
    seqs = seqs.astype(jnp.int32)
    query_tok = query_tok.astype(jnp.int32)
    ts2d, wcat = _prep(embed, sW1, sb1.reshape(1, -1), sW2, sb2.reshape(1, -1), rW1)
    counts = _sim(seqs, ts2d.reshape(VOCAB))
    return _head(counts, query_tok, wcat, rb1.reshape(1, -1), rW2, rb2.reshape(1, -1))


# speedup vs baseline: 583.2343x; 1.8978x over previous
import functools

import jax
import jax.numpy as jnp
from jax import lax
from jax.experimental import pallas as pl
from jax.experimental.pallas import tpu as pltpu
from jax.experimental.pallas import tpu_sc as plsc

HID = 64
VOCAB = 64
SLOTS = 8
SEQ = 32
LANES = 16
NW = 32


def _prep_body(embT_ref, sW1T_ref, sb1_ref, sW2T_ref, sb2_ref, rW1T_ref,
               ts_ref, wcatT_ref):
    embT = embT_ref[:, 0:VOCAB]
    hT = jnp.maximum(jnp.dot(sW1T_ref[...], embT) + sb1_ref[...], 0.0)
    ts_ref[...] = jnp.dot(sW2T_ref[...], hT) + sb2_ref[...]
    wcatT_ref[:, 0:VOCAB] = jnp.dot(rW1T_ref[:, 0:HID], embT)
    wcatT_ref[:, VOCAB:2 * VOCAB] = (
        jnp.dot(rW1T_ref[:, HID:2 * HID], embT) * (1.0 / SLOTS))


def _prep(embed, sW1, sb1, sW2, sb2, rW1):
    return pl.pallas_call(
        _prep_body,
        out_shape=(
            jax.ShapeDtypeStruct((1, VOCAB), jnp.float32),
            jax.ShapeDtypeStruct((HID, 2 * VOCAB), jnp.float32),
        ),
    )(embed.T, sW1.T, sb1.reshape(-1, 1), sW2.T, sb2.reshape(-1, 1), rW1.T)


def _sim_body(seqsT_hbm, ts_hbm, cntT_hbm, ts_v, seq_v, cnt_v):
    wid = lax.axis_index("s") * 2 + lax.axis_index("c")
    per_w = seqsT_hbm.shape[1] // NW
    groups = per_w // LANES
    base = wid * per_w

    pltpu.sync_copy(ts_hbm, ts_v)
    pltpu.sync_copy(seqsT_hbm.at[:, pl.ds(base, per_w)], seq_v)

    zeros_f = jnp.zeros((LANES,), jnp.float32)
    ones_f = jnp.full((LANES,), 1.0, jnp.float32)
    lane = lax.iota(jnp.int32, LANES)

    def one_group(g):
        samp = lane + g * LANES

        for r in range(VOCAB):
            cnt_v[r, pl.ds(g * LANES, LANES)] = zeros_f

        tok = []
        sc = []
        for t in range(SLOTS):
            tt = plsc.load_gather(seq_v, [jnp.full((LANES,), t, jnp.int32), samp])
            tok.append(tt)
            sc.append(plsc.load_gather(ts_v, [tt]))

        for t in range(SLOTS, SEQ - 1):
            ntok = plsc.load_gather(seq_v, [jnp.full((LANES,), t, jnp.int32), samp])
            nsc = plsc.load_gather(ts_v, [ntok])
            pref = [sc[0]]
            for i in range(1, SLOTS - 1):
                pref.append(jnp.minimum(pref[-1], sc[i]))
            m = jnp.minimum(pref[-1], sc[SLOTS - 1])
            ntoks = []
            nscs = []
            for i in range(SLOTS - 1):
                keep = pref[i] > m
                ntoks.append(jnp.where(keep, tok[i], tok[i + 1]))
                nscs.append(jnp.where(keep, sc[i], sc[i + 1]))
            tok = ntoks + [ntok]
            sc = nscs + [nsc]

        for i in range(SLOTS):
            plsc.addupdate_scatter(cnt_v, [tok[i], samp], ones_f)

    def group_pair(gg, carry):
        one_group(gg * 2)
        one_group(gg * 2 + 1)
        return carry

    lax.fori_loop(0, groups // 2, group_pair, 0)
    pltpu.sync_copy(cnt_v, cntT_hbm.at[:, pl.ds(base, per_w)])


def _sim(seqsT, ts):
    B = seqsT.shape[1]
    mesh = plsc.VectorSubcoreMesh(core_axis_name="c", subcore_axis_name="s")
    per_w = B // NW
    f = functools.partial(
        pl.kernel,
        out_type=jax.ShapeDtypeStruct((VOCAB, B), jnp.float32),
        mesh=mesh,
        scratch_types=[
            pltpu.VMEM((VOCAB,), jnp.float32),
            pltpu.VMEM((SEQ, per_w), jnp.int32),
            pltpu.VMEM((VOCAB, per_w), jnp.float32),
        ],
        compiler_params=pltpu.CompilerParams(
            needs_layout_passes=False, use_tc_tiling_on_sc=False),
    )(_sim_body)
    return f(seqsT, ts)


def _head_body(cntT_ref, q_ref, wcatT_ref, rb1_ref, rW2T_ref, rb2_ref, out_ref):
    blk = cntT_ref.shape[1]
    iota = lax.broadcasted_iota(jnp.int32, (VOCAB, blk), 0)
    qohT = (q_ref[...] == iota).astype(jnp.float32)
    hT = (jnp.dot(wcatT_ref[:, 0:VOCAB], qohT)
          + jnp.dot(wcatT_ref[:, VOCAB:2 * VOCAB], cntT_ref[...])
          + rb1_ref[...])
    hT = jnp.maximum(hT, 0.0)
    out_ref[...] = jnp.dot(rW2T_ref[...], hT) + rb2_ref[...]


def _head(cntT, query_tok, wcatT, rb1, rW2, rb2):
    B = cntT.shape[1]
    blk = 4096
    outT = pl.pallas_call(
        _head_body,
        grid=(B // blk,),
        in_specs=[
            pl.BlockSpec((VOCAB, blk), lambda i: (0, i)),
            pl.BlockSpec((1, blk), lambda i: (0, i)),
            pl.BlockSpec((HID, 2 * VOCAB), lambda i: (0, 0)),
            pl.BlockSpec((HID, 1), lambda i: (0, 0)),
            pl.BlockSpec((HID, VOCAB), lambda i: (0, 0)),
            pl.BlockSpec((VOCAB, 1), lambda i: (0, 0)),
        ],
        out_specs=pl.BlockSpec((VOCAB, blk), lambda i: (0, i)),
        out_shape=jax.ShapeDtypeStruct((VOCAB, B), jnp.float32),
        compiler_params=pltpu.CompilerParams(
            dimension_semantics=("parallel",)),
    )(cntT, query_tok.reshape(1, B), wcatT, rb1.reshape(-1, 1),
      rW2.T, rb2.reshape(-1, 1))
    return outT.T


def kernel(seqs, query_tok, embed, sW1, sb1, sW2, sb2, rW1, rb1, rW2, rb2):
    seqs = seqs.astype(jnp.int32)
    query_tok = query_tok.astype(jnp.int32)
    tsT, wcatT = _prep(embed, sW1, sb1, sW2, sb2, rW1)
    cntT = _sim(seqs.T, tsT.reshape(VOCAB))
    return _head(cntT, query_tok, wcatT, rb1, rW2, rb2)
